# baseline (speedup 1.0000x reference)
import jax
import jax.numpy as jnp
from jax.experimental import pallas as pl
from jax.experimental.pallas import tpu as pltpu

D_MODEL = 1024
NUM_EXPERTS = 16
HIDDEN = 256
N_TOK = 8192
BLK = 2048
OUT_SPLIT = 2
OUT_BLK = D_MODEL // OUT_SPLIT


def _moe_block(x_ref, Wg_ref, bg_ref, W1_ref, b1_ref, W2_ref, b2_ref,
               W3_ref, b3_ref, o_ref, h2_ref, gate_ref):
    @pl.when(pl.program_id(1) == 0)
    def _head():
        xb = x_ref[...]

        logits = jnp.dot(xb, Wg_ref[...], preferred_element_type=jnp.float32)
        logits = logits + bg_ref[...][None, :]
        m = jnp.max(logits, axis=-1, keepdims=True)
        e = jnp.exp(logits - m)
        denom = jnp.sum(e, axis=-1)
        pos = jax.lax.broadcasted_iota(jnp.int32, e.shape, 1)
        first = jnp.min(jnp.where(logits == m, pos, NUM_EXPERTS), axis=-1)
        m2 = jnp.max(jnp.where(pos == first[:, None], 0.0, e), axis=-1)
        gate_ref[...] = (1.0 + m2) / denom

        h = jnp.dot(xb, W1_ref[...], preferred_element_type=jnp.float32)
        h = jnp.maximum(h + b1_ref[...][None, :], 0.0)
        h = jnp.dot(h, W2_ref[...], preferred_element_type=jnp.float32)
        h2_ref[...] = jnp.maximum(h + b2_ref[...][None, :], 0.0)

    h = jnp.dot(h2_ref[...], W3_ref[...], preferred_element_type=jnp.float32)
    h = jnp.maximum(h + b3_ref[...][None, :], 0.0)
    o_ref[...] = h * gate_ref[...][:, None]


def kernel(x, Wg, bg, W1, b1, W2, b2, W3, b3):
    n = x.shape[0]
    grid = (n // BLK, OUT_SPLIT)
    full = lambda *shape: pl.BlockSpec(shape, lambda i, j: (0,) * len(shape))
    return pl.pallas_call(
        _moe_block,
        grid=grid,
        in_specs=[
            pl.BlockSpec((BLK, D_MODEL), lambda i, j: (i, 0)),
            full(D_MODEL, NUM_EXPERTS),
            full(NUM_EXPERTS),
            full(D_MODEL, HIDDEN),
            full(HIDDEN),
            full(HIDDEN, HIDDEN),
            full(HIDDEN),
            pl.BlockSpec((HIDDEN, OUT_BLK), lambda i, j: (0, j)),
            pl.BlockSpec((OUT_BLK,), lambda i, j: (j,)),
        ],
        out_specs=pl.BlockSpec((BLK, OUT_BLK), lambda i, j: (i, j)),
        out_shape=jax.ShapeDtypeStruct((n, D_MODEL), jnp.float32),
        scratch_shapes=[
            pltpu.VMEM((BLK, HIDDEN), jnp.float32),
            pltpu.VMEM((BLK,), jnp.float32),
        ],
        compiler_params=pltpu.CompilerParams(
            dimension_semantics=("parallel", "arbitrary")),
    )(x, Wg, bg, W1, b1, W2, b2, W3, b3)


# speedup vs baseline: 1.9078x; 1.9078x over previous
import jax
import jax.numpy as jnp
from jax.experimental import pallas as pl
from jax.experimental.pallas import tpu as pltpu

D_MODEL = 1024
NUM_EXPERTS = 16
HIDDEN = 256
N_TOK = 8192
BLK = 2048


def _moe_block(x_ref, Wg_ref, bg_ref, W1_ref, b1_ref, W2_ref, b2_ref,
               W3_ref, b3_ref, o_ref):
    xb = x_ref[...]

    logits = jnp.dot(xb, Wg_ref[...], preferred_element_type=jnp.float32)
    logits = logits + bg_ref[...][None, :]
    m = jnp.max(logits, axis=-1, keepdims=True)
    e = jnp.exp(logits - m)
    denom = jnp.sum(e, axis=-1)
    pos = jax.lax.broadcasted_iota(jnp.int32, e.shape, 1)
    first = jnp.min(jnp.where(logits == m, pos, NUM_EXPERTS), axis=-1)
    m2 = jnp.max(jnp.where(pos == first[:, None], 0.0, e), axis=-1)
    gate = (1.0 + m2) / denom

    h = jnp.dot(xb, W1_ref[...], preferred_element_type=jnp.float32)
    h = jnp.maximum(h + b1_ref[...][None, :], 0.0)
    h = jnp.dot(h, W2_ref[...], preferred_element_type=jnp.float32)
    h = jnp.maximum(h + b2_ref[...][None, :], 0.0)
    h = jnp.dot(h, W3_ref[...], preferred_element_type=jnp.float32)
    h = jnp.maximum(h + b3_ref[...][None, :], 0.0)

    o_ref[...] = h * gate[:, None]


def kernel(x, Wg, bg, W1, b1, W2, b2, W3, b3):
    n = x.shape[0]
    grid = (n // BLK,)
    full = lambda *shape: pl.BlockSpec(shape, lambda i: (0,) * len(shape))
    return pl.pallas_call(
        _moe_block,
        grid=grid,
        in_specs=[
            pl.BlockSpec((BLK, D_MODEL), lambda i: (i, 0)),
            full(D_MODEL, NUM_EXPERTS),
            full(NUM_EXPERTS),
            full(D_MODEL, HIDDEN),
            full(HIDDEN),
            full(HIDDEN, HIDDEN),
            full(HIDDEN),
            full(HIDDEN, D_MODEL),
            full(D_MODEL),
        ],
        out_specs=pl.BlockSpec((BLK, D_MODEL), lambda i: (i, 0)),
        out_shape=jax.ShapeDtypeStruct((n, D_MODEL), jnp.float32),
        compiler_params=pltpu.CompilerParams(
            dimension_semantics=("parallel",)),
    )(x, Wg, bg, W1, b1, W2, b2, W3, b3)
